# baseline (speedup 1.0000x reference)
import functools

import jax
import jax.numpy as jnp
from jax import lax
from jax.experimental import pallas as pl
from jax.experimental.pallas import tpu as pltpu
from jax.experimental.pallas import tpu_sc as plsc

NUM_FIELDS = 26
VOCAB = 100000
D = 16
B = 16384

NC = 2
NS = 16
NW = NC * NS
BPW = B // NW

_mesh = plsc.VectorSubcoreMesh(
    core_axis_name="c", subcore_axis_name="s", num_cores=NC, num_subcores=NS
)


@functools.partial(
    pl.kernel,
    out_type=jax.ShapeDtypeStruct((B, NUM_FIELDS * D), jnp.float32),
    mesh=_mesh,
    scratch_types=[
        pltpu.VMEM((BPW,), jnp.int32),
        pltpu.VMEM((BPW, D), jnp.float32),
        pltpu.SemaphoreType.DMA,
    ],
    compiler_params=pltpu.CompilerParams(use_tc_tiling_on_sc=False),
)
def _embed_cat(*refs):
    feats = refs[:NUM_FIELDS]
    tables = refs[NUM_FIELDS:2 * NUM_FIELDS]
    out = refs[2 * NUM_FIELDS]
    idx_v, rows_v, sem = refs[2 * NUM_FIELDS + 1:]

    wid = lax.axis_index("s") * NC + lax.axis_index("c")
    base = wid * BPW

    for f in range(NUM_FIELDS):
        pltpu.sync_copy(feats[f].at[pl.ds(base, BPW)], idx_v)
        pltpu.async_copy(tables[f].at[idx_v], rows_v, sem).wait()
        pltpu.sync_copy(rows_v, out.at[pl.ds(base, BPW), pl.ds(f * D, D)])


def kernel(feat_0, feat_1, feat_2, feat_3, feat_4, feat_5, feat_6, feat_7, feat_8, feat_9, feat_10, feat_11, feat_12, feat_13, feat_14, feat_15, feat_16, feat_17, feat_18, feat_19, feat_20, feat_21, feat_22, feat_23, feat_24, feat_25, table_0, table_1, table_2, table_3, table_4, table_5, table_6, table_7, table_8, table_9, table_10, table_11, table_12, table_13, table_14, table_15, table_16, table_17, table_18, table_19, table_20, table_21, table_22, table_23, table_24, table_25):
    args = locals()
    feats = [args[f"feat_{i}"] for i in range(NUM_FIELDS)]
    tables = [args[f"table_{i}"] for i in range(NUM_FIELDS)]
    return _embed_cat(*feats, *tables)
